# baseline (speedup 1.0000x reference)
import functools

import jax
import jax.numpy as jnp
from jax import lax
from jax.experimental import pallas as pl
from jax.experimental.pallas import tpu as pltpu
from jax.experimental.pallas import tpu_sc as plsc

_B = 16384
_F = 32
_NC = 2
_NS = 16
_NW = _NC * _NS
_BPW = _B // _NW
_CHUNK = 128
_NCHUNK = _BPW // _CHUNK


def _make_mf():
    mesh = plsc.VectorSubcoreMesh(core_axis_name="c", subcore_axis_name="s")

    @functools.partial(
        pl.kernel,
        mesh=mesh,
        out_type=jax.ShapeDtypeStruct((_B,), jnp.float32),
        compiler_params=pltpu.CompilerParams(
            needs_layout_passes=False, use_tc_tiling_on_sc=False),
        scratch_types=[
            pltpu.VMEM((_NCHUNK, _CHUNK), jnp.int32),
            pltpu.VMEM((_NCHUNK, _CHUNK), jnp.int32),
            pltpu.VMEM((_BPW, _F), jnp.float32),
            pltpu.VMEM((_BPW, _F), jnp.float32),
            pltpu.VMEM((_BPW,), jnp.float32),
            pltpu.VMEM((_F,), jnp.float32),
            pltpu.VMEM((16,), jnp.float32),
            pltpu.SemaphoreType.DMA,
            pltpu.SemaphoreType.DMA,
        ],
    )
    def mf(u_hbm, i_hbm, ue_hbm, ie_hbm, w_hbm, b_hbm, out_hbm,
           uidx, iidx, urows, irows, outv, wv, bv, sem_u, sem_i):
        wid = lax.axis_index("s") * _NC + lax.axis_index("c")
        crow = wid * _NCHUNK
        pltpu.sync_copy(u_hbm.at[pl.ds(crow, _NCHUNK)], uidx)
        pltpu.sync_copy(i_hbm.at[pl.ds(crow, _NCHUNK)], iidx)
        pltpu.sync_copy(w_hbm, wv)
        pltpu.sync_copy(b_hbm, bv)

        copies = []
        for c in range(_NCHUNK):
            copies.append(pltpu.async_copy(
                ue_hbm.at[uidx.at[c]],
                urows.at[pl.ds(c * _CHUNK, _CHUNK)], sem_u))
            copies.append(pltpu.async_copy(
                ie_hbm.at[iidx.at[c]],
                irows.at[pl.ds(c * _CHUNK, _CHUNK)], sem_i))
        for cp in copies:
            cp.wait()


        iota = lax.iota(jnp.int32, 16)
        w_lo = wv[pl.ds(0, 16)]
        w_hi = wv[pl.ds(16, 16)]
        b0 = bv[pl.ds(0, 16)][0]

        def body(g, carry):
            rid = g * 16 + iota
            acc = jnp.zeros((16,), jnp.float32)
            for f in range(_F):
                fidx = jnp.full((16,), f, jnp.int32)
                cu = plsc.load_gather(urows, [rid, fidx])
                ci = plsc.load_gather(irows, [rid, fidx])
                wf = w_lo[f] if f < 16 else w_hi[f - 16]
                acc = acc + cu * ci * wf
            outv[pl.ds(g * 16, 16)] = jnp.maximum(acc + b0, 0.0)
            return carry

        lax.fori_loop(0, _BPW // 16, body, 0, unroll=2)

        pltpu.sync_copy(outv, out_hbm.at[pl.ds(wid * _BPW, _BPW)])

    return mf


_mf = _make_mf()


def kernel(u, i, users_emb, items_emb, W, b):
    u2 = u.reshape(_NW * _NCHUNK, _CHUNK)
    i2 = i.reshape(_NW * _NCHUNK, _CHUNK)
    w = W.reshape(_F)
    bp = jnp.pad(b, (0, 15))
    out = _mf(u2, i2, users_emb, items_emb, w, bp)
    return out.reshape(_B, 1)


# speedup vs baseline: 2.0482x; 2.0482x over previous
import functools

import jax
import jax.numpy as jnp
from jax import lax
from jax.experimental import pallas as pl
from jax.experimental.pallas import tpu as pltpu
from jax.experimental.pallas import tpu_sc as plsc

_B = 16384
_F = 32
_NC = 2
_NS = 16
_NW = _NC * _NS
_BPW = _B // _NW
_CHUNK = 32
_NCH = _BPW // _CHUNK


def _make_mf():
    mesh = plsc.VectorSubcoreMesh(core_axis_name="c", subcore_axis_name="s")

    @functools.partial(
        pl.kernel,
        mesh=mesh,
        out_type=jax.ShapeDtypeStruct((_B,), jnp.float32),
        compiler_params=pltpu.CompilerParams(
            needs_layout_passes=False, use_tc_tiling_on_sc=True),
        scratch_types=[
            pltpu.VMEM((_BPW,), jnp.int32),
            pltpu.VMEM((_BPW,), jnp.int32),
            pltpu.VMEM((_BPW,), jnp.int32),
            pltpu.VMEM((_BPW,), jnp.int32),
            pltpu.VMEM((_CHUNK, 8, _F), jnp.float32),
            pltpu.VMEM((_CHUNK, 8, _F), jnp.float32),
            pltpu.VMEM((_BPW,), jnp.float32),
            pltpu.VMEM((_F,), jnp.float32),
            pltpu.VMEM((16,), jnp.float32),
            pltpu.SemaphoreType.DMA,
            pltpu.SemaphoreType.DMA,
        ],
    )
    def mf(u_hbm, i_hbm, ue_hbm, ie_hbm, w_hbm, b_hbm, out_hbm,
           uidx, iidx, su, si, ubuf, ibuf, outv, wv, bv,
           sem_u, sem_i):
        wid = lax.axis_index("s") * _NC + lax.axis_index("c")
        base = wid * _BPW
        pltpu.sync_copy(u_hbm.at[pl.ds(base, _BPW)], uidx)
        pltpu.sync_copy(i_hbm.at[pl.ds(base, _BPW)], iidx)
        pltpu.sync_copy(w_hbm, wv)
        pltpu.sync_copy(b_hbm, bv)

        iota = lax.iota(jnp.int32, 16)
        w_lo = wv[pl.ds(0, 16)]
        w_hi = wv[pl.ds(16, 16)]
        b0 = bv[pl.ds(0, 16)][0]

        def split(q, carry):
            o = q * 16
            su[pl.ds(o, 16)] = lax.bitwise_and(uidx[pl.ds(o, 16)], 7)
            si[pl.ds(o, 16)] = lax.bitwise_and(iidx[pl.ds(o, 16)], 7)
            return carry

        lax.fori_loop(0, _BPW // 16, split, 0, unroll=2)

        def chunk(c, carry):
            o = c * _CHUNK
            for blk in range(_CHUNK // 16):
                tv_u = lax.shift_right_logical(
                    uidx[pl.ds(o + blk * 16, 16)], 3)
                tv_i = lax.shift_right_logical(
                    iidx[pl.ds(o + blk * 16, 16)], 3)
                for j in range(16):
                    k = blk * 16 + j
                    pltpu.async_copy(ue_hbm.at[tv_u[j]], ubuf.at[k], sem_u)
                    pltpu.async_copy(ie_hbm.at[tv_i[j]], ibuf.at[k], sem_i)
            pltpu.make_async_copy(ue_hbm.at[pl.ds(0, _CHUNK)], ubuf,
                                  sem_u).wait()
            pltpu.make_async_copy(ie_hbm.at[pl.ds(0, _CHUNK)], ibuf,
                                  sem_i).wait()

            for blk in range(_CHUNK // 16):
                kvec = blk * 16 + iota
                sv_u = su[pl.ds(o + blk * 16, 16)]
                sv_i = si[pl.ds(o + blk * 16, 16)]
                acc = jnp.zeros((16,), jnp.float32)
                for f in range(_F):
                    fvec = jnp.full((16,), f, jnp.int32)
                    cu = plsc.load_gather(ubuf, [kvec, sv_u, fvec])
                    ci = plsc.load_gather(ibuf, [kvec, sv_i, fvec])
                    wf = w_lo[f] if f < 16 else w_hi[f - 16]
                    acc = acc + cu * ci * wf
                outv[pl.ds(o + blk * 16, 16)] = jnp.maximum(acc + b0, 0.0)
            return carry

        lax.fori_loop(0, _NCH, chunk, 0)

        pltpu.sync_copy(outv, out_hbm.at[pl.ds(base, _BPW)])

    return mf


_mf = _make_mf()


def kernel(u, i, users_emb, items_emb, W, b):
    ue3 = users_emb.reshape(-1, 8, _F)
    ie3 = items_emb.reshape(-1, 8, _F)
    w = W.reshape(_F)
    bp = jnp.pad(b, (0, 15))
    out = _mf(u, i, ue3, ie3, w, bp)
    return out.reshape(_B, 1)
